# baseline (speedup 1.0000x reference)
import functools

import jax
import jax.numpy as jnp
from jax import lax
from jax.experimental import pallas as pl
from jax.experimental.pallas import tpu as pltpu
from jax.experimental.pallas import tpu_sc as plsc

NC = 2
NS = 16
L = 16
SECG = 12


def _gamma_beta_body(cids_ref, cond_ref, wh_ref, wg_ref, wb_ref, g0_ref,
                     b0_ref, gamma_ref, beta_ref):
    npad = cids_ref.shape[0]
    oh = (cids_ref[...] == lax.broadcasted_iota(jnp.int32, (npad, npad), 1))
    oh = oh.astype(jnp.float32)
    dn = (((1,), (1,)), ((), ()))
    t1 = lax.dot_general(cond_ref[...], wh_ref[...], dn,
                         preferred_element_type=jnp.float32)
    g_all = lax.dot_general(t1, wg_ref[...], dn,
                            preferred_element_type=jnp.float32)
    b_all = lax.dot_general(t1, wb_ref[...], dn,
                            preferred_element_type=jnp.float32)
    gamma_ref[...] = g0_ref[...] + jnp.dot(
        oh, g_all, preferred_element_type=jnp.float32)
    beta_ref[...] = b0_ref[...] + jnp.dot(
        oh, b_all, preferred_element_type=jnp.float32)


def _make_sc_embed(nb, s, d, npad):
    nw = NC * NS
    spw = s // nw
    ch = spw // 2
    nch = 2 * nb
    nin = 2
    nout = 2
    ng = d // L
    nsec = ng // SECG
    mesh = plsc.VectorSubcoreMesh(core_axis_name="c", subcore_axis_name="s")

    @functools.partial(
        pl.kernel,
        mesh=mesh,
        out_type=jax.ShapeDtypeStruct((nb * s, d), jnp.float32),
        scratch_types=[
            pltpu.VMEM((nb * spw,), jnp.int32),
            pltpu.VMEM((nin, ch, d), jnp.float32),
            pltpu.VMEM((nout, ch, d), jnp.float32),
            pltpu.VMEM((ch, d), jnp.float32),
            pltpu.VMEM((nb, d), jnp.float32),
            pltpu.VMEM((nb, d), jnp.float32),
            pltpu.SemaphoreType.DMA,
            pltpu.SemaphoreType.DMA,
            pltpu.SemaphoreType.DMA,
            pltpu.SemaphoreType.DMA,
            pltpu.SemaphoreType.DMA,
            pltpu.SemaphoreType.DMA,
            pltpu.SemaphoreType.DMA,
        ],
    )
    def sc_embed(ids_hbm, word_hbm, pos_hbm, gamma_hbm, beta_hbm, out_hbm,
                 ids_v, rin_v, rout_v, pos_v, gam_v, bet_v,
                 g0, g1, g2, o0, o1, isem, psem):
        wid = lax.axis_index("s") * NC + lax.axis_index("c")
        p0 = wid * spw
        id_cps = [pltpu.async_copy(ids_hbm.at[pl.ds(b * s + p0, spw)],
                                   ids_v.at[pl.ds(b * spw, spw)], isem)
                  for b in range(nb)]
        p_cps = [
            pltpu.async_copy(pos_hbm.at[pl.ds(p0, ch)], pos_v, psem),
            pltpu.async_copy(gamma_hbm.at[pl.ds(0, nb)], gam_v, psem),
            pltpu.async_copy(beta_hbm.at[pl.ds(0, nb)], bet_v, psem),
        ]
        for cp in id_cps:
            cp.wait()

        gsem = (g0, g1, g2)
        osem = (o0, o1)

        def start_gather(c):
            si, b = c // nb, c % nb
            return pltpu.async_copy(
                word_hbm.at[ids_v.at[pl.ds(b * spw + si * ch, ch)]],
                rin_v.at[c % nin], gsem[c % nin])

        gathers = [None] * nch
        outs = [None] * nch
        for c in range(nin - 1):
            gathers[c] = start_gather(c)
        for cp in p_cps:
            cp.wait()
        for c in range(nch):
            si, b = c // nb, c % nb
            if c == nb:
                pltpu.sync_copy(pos_hbm.at[pl.ds(p0 + ch, ch)], pos_v)
            if c + nin - 1 < nch:
                gathers[c + nin - 1] = start_gather(c + nin - 1)
            gathers[c].wait()
            if c >= nout:
                outs[c - nout].wait()
            rbuf = rin_v.at[c % nin]
            obuf = rout_v.at[c % nout]

            def sec_body(sec, carry, _b=b, _rbuf=rbuf, _obuf=obuf):
                base = sec * (SECG * L)
                gg = [gam_v[_b, pl.ds(base + j * L, L)] for j in range(SECG)]
                bb = [bet_v[_b, pl.ds(base + j * L, L)] for j in range(SECG)]

                def body(r, carry2):
                    for j in range(SECG):
                        sl = pl.ds(base + j * L, L)
                        _obuf[r, sl] = (
                            (_rbuf[r, sl] + pos_v[r, sl])
                            * gg[j] + bb[j])
                    return carry2

                lax.fori_loop(0, ch, body, 0)
                return carry

            lax.fori_loop(0, nsec, sec_body, 0)
            outs[c] = pltpu.async_copy(
                obuf, out_hbm.at[pl.ds(b * s + p0 + si * ch, ch)],
                osem[c % nout])
        for c in range(nch - nout, nch):
            outs[c].wait()

    return sc_embed


def kernel(input_ids, condition_ids, word_emb, pos_emb, cond_emb, W_hidden,
           W_gma, W_bta, clngma, clnbta):
    B, S = input_ids.shape
    vocab, D = word_emb.shape
    cond_size = cond_emb.shape[0]
    npad = 8

    cids_p = jnp.zeros((npad, 1), jnp.int32).at[:B, 0].set(
        condition_ids.astype(jnp.int32))
    cond_p = jnp.zeros((npad, cond_emb.shape[1]), jnp.float32).at[
        :cond_size].set(cond_emb)
    gamma, beta = pl.pallas_call(
        _gamma_beta_body,
        out_shape=(
            jax.ShapeDtypeStruct((npad, D), jnp.float32),
            jax.ShapeDtypeStruct((npad, D), jnp.float32),
        ),
    )(cids_p, cond_p, W_hidden, W_gma, W_bta,
      clngma.reshape(1, D), clnbta.reshape(1, D))

    ids_flat = input_ids.reshape(B * S).astype(jnp.int32)
    sc_embed = _make_sc_embed(B, S, D, npad)
    out = sc_embed(ids_flat, word_emb, pos_emb, gamma, beta)
    return out.reshape(B, S, D)


# speedup vs baseline: 2.3110x; 2.3110x over previous
import functools

import jax
import jax.numpy as jnp
from jax import lax
from jax.experimental import pallas as pl
from jax.experimental.pallas import tpu as pltpu
from jax.experimental.pallas import tpu_sc as plsc

NC = 2
NS = 16
L = 16
SECG = 12


def _gamma_beta_body(cids_ref, cond_ref, wh_ref, wg_ref, wb_ref, g0_ref,
                     b0_ref, gamma_ref, beta_ref):
    nb = cids_ref.shape[0]
    npad = cond_ref.shape[0]
    row = lax.broadcasted_iota(jnp.int32, (npad, npad), 0)
    col = lax.broadcasted_iota(jnp.int32, (npad, npad), 1)
    oh = jnp.zeros((npad, npad), jnp.float32)
    for b in range(nb):
        oh = oh + jnp.where((row == b) & (col == cids_ref[b]), 1.0, 0.0)
    dn = (((1,), (1,)), ((), ()))
    t1 = lax.dot_general(cond_ref[...], wh_ref[...], dn,
                         preferred_element_type=jnp.float32)
    g_all = lax.dot_general(t1, wg_ref[...], dn,
                            preferred_element_type=jnp.float32)
    b_all = lax.dot_general(t1, wb_ref[...], dn,
                            preferred_element_type=jnp.float32)
    gamma_ref[...] = g0_ref[...] + jnp.dot(
        oh, g_all, preferred_element_type=jnp.float32)
    beta_ref[...] = b0_ref[...] + jnp.dot(
        oh, b_all, preferred_element_type=jnp.float32)


def _make_sc_embed(nb, s, d, npad):
    nw = NC * NS
    spw = s // nw
    ch = spw // 2
    nch = 2 * nb
    nin = 2
    nout = 2
    ng = d // L
    nsec = ng // SECG
    mesh = plsc.VectorSubcoreMesh(core_axis_name="c", subcore_axis_name="s")

    @functools.partial(
        pl.kernel,
        mesh=mesh,
        out_type=jax.ShapeDtypeStruct((nb * s, d), jnp.float32),
        scratch_types=[
            pltpu.VMEM((nb * spw,), jnp.int32),
            pltpu.VMEM((nin, ch, d), jnp.float32),
            pltpu.VMEM((nout, ch, d), jnp.float32),
            pltpu.VMEM((ch, d), jnp.float32),
            pltpu.VMEM((nb, d), jnp.float32),
            pltpu.VMEM((nb, d), jnp.float32),
            pltpu.SemaphoreType.DMA,
            pltpu.SemaphoreType.DMA,
            pltpu.SemaphoreType.DMA,
            pltpu.SemaphoreType.DMA,
            pltpu.SemaphoreType.DMA,
            pltpu.SemaphoreType.DMA,
            pltpu.SemaphoreType.DMA,
        ],
    )
    def sc_embed(ids_hbm, word_hbm, pos_hbm, gamma_hbm, beta_hbm, out_hbm,
                 ids_v, rin_v, rout_v, pos_v, gam_v, bet_v,
                 g0, g1, g2, o0, o1, isem, psem):
        wid = lax.axis_index("s") * NC + lax.axis_index("c")
        p0 = wid * spw
        id_cps = [pltpu.async_copy(ids_hbm.at[pl.ds(b * s + p0, spw)],
                                   ids_v.at[pl.ds(b * spw, spw)], isem)
                  for b in range(nb)]
        p_cps = [
            pltpu.async_copy(pos_hbm.at[pl.ds(p0, ch)], pos_v, psem),
            pltpu.async_copy(gamma_hbm.at[pl.ds(0, nb)], gam_v, psem),
            pltpu.async_copy(beta_hbm.at[pl.ds(0, nb)], bet_v, psem),
        ]
        for cp in id_cps:
            cp.wait()

        gsem = (g0, g1, g2)
        osem = (o0, o1)

        def start_gather(c):
            si, b = c // nb, c % nb
            return pltpu.async_copy(
                word_hbm.at[ids_v.at[pl.ds(b * spw + si * ch, ch)]],
                rin_v.at[c % nin], gsem[c % nin])

        gathers = [None] * nch
        outs = [None] * nch
        for c in range(nin - 1):
            gathers[c] = start_gather(c)
        for cp in p_cps:
            cp.wait()
        for c in range(nch):
            si, b = c // nb, c % nb
            if c == nb:
                pltpu.sync_copy(pos_hbm.at[pl.ds(p0 + ch, ch)], pos_v)
            if c + nin - 1 < nch:
                gathers[c + nin - 1] = start_gather(c + nin - 1)
            gathers[c].wait()
            if c >= nout:
                outs[c - nout].wait()
            rbuf = rin_v.at[c % nin]
            obuf = rout_v.at[c % nout]
            for sec in range(nsec):
                gg = [gam_v[b, pl.ds((sec * SECG + j) * L, L)]
                      for j in range(SECG)]
                bb = [bet_v[b, pl.ds((sec * SECG + j) * L, L)]
                      for j in range(SECG)]

                def body(r, carry, _sec=sec, _gg=gg, _bb=bb,
                         _rbuf=rbuf, _obuf=obuf):
                    for j in range(SECG):
                        sl = pl.ds((_sec * SECG + j) * L, L)
                        _obuf[r, sl] = (
                            (_rbuf[r, sl] + pos_v[r, sl])
                            * _gg[j] + _bb[j])
                    return carry

                lax.fori_loop(0, ch, body, 0)
            outs[c] = pltpu.async_copy(
                obuf, out_hbm.at[pl.ds(b * s + p0 + si * ch, ch)],
                osem[c % nout])
        for c in range(nch - nout, nch):
            outs[c].wait()

    return sc_embed


def kernel(input_ids, condition_ids, word_emb, pos_emb, cond_emb, W_hidden,
           W_gma, W_bta, clngma, clnbta):
    B, S = input_ids.shape
    vocab, D = word_emb.shape
    cond_size = cond_emb.shape[0]
    npad = 8

    cond_p = jnp.zeros((npad, cond_emb.shape[1]), jnp.float32).at[
        :cond_size].set(cond_emb)
    gamma, beta = pl.pallas_call(
        _gamma_beta_body,
        in_specs=[
            pl.BlockSpec(memory_space=pltpu.SMEM),
            pl.BlockSpec(memory_space=pltpu.VMEM),
            pl.BlockSpec(memory_space=pltpu.VMEM),
            pl.BlockSpec(memory_space=pltpu.VMEM),
            pl.BlockSpec(memory_space=pltpu.VMEM),
            pl.BlockSpec(memory_space=pltpu.VMEM),
            pl.BlockSpec(memory_space=pltpu.VMEM),
        ],
        out_shape=(
            jax.ShapeDtypeStruct((npad, D), jnp.float32),
            jax.ShapeDtypeStruct((npad, D), jnp.float32),
        ),
    )(condition_ids.astype(jnp.int32), cond_p, W_hidden, W_gma, W_bta,
      clngma.reshape(1, D), clnbta.reshape(1, D))

    ids_flat = input_ids.reshape(B * S).astype(jnp.int32)
    sc_embed = _make_sc_embed(B, S, D, npad)
    out = sc_embed(ids_flat, word_emb, pos_emb, gamma, beta)
    return out.reshape(B, S, D)


# speedup vs baseline: 2.3691x; 1.0252x over previous
import functools

import jax
import jax.numpy as jnp
from jax import lax
from jax.experimental import pallas as pl
from jax.experimental.pallas import tpu as pltpu
from jax.experimental.pallas import tpu_sc as plsc

NC = 2
NS = 16
L = 16
SECG = 12


def _gamma_beta_body(cids_ref, cond_ref, wh_ref, wg_ref, wb_ref, g0_ref,
                     b0_ref, gamma_ref, beta_ref):
    nb = cids_ref.shape[0]
    npad = 8
    cond_p = jnp.concatenate(
        [cond_ref[...],
         jnp.zeros((npad - cond_ref.shape[0], cond_ref.shape[1]),
                   jnp.float32)], axis=0)
    row = lax.broadcasted_iota(jnp.int32, (npad, npad), 0)
    col = lax.broadcasted_iota(jnp.int32, (npad, npad), 1)
    oh = jnp.zeros((npad, npad), jnp.float32)
    for b in range(nb):
        oh = oh + jnp.where((row == b) & (col == cids_ref[b]), 1.0, 0.0)
    dn = (((1,), (1,)), ((), ()))
    t1 = lax.dot_general(cond_p, wh_ref[...], dn,
                         preferred_element_type=jnp.float32)
    g_all = lax.dot_general(t1, wg_ref[...], dn,
                            preferred_element_type=jnp.float32)
    b_all = lax.dot_general(t1, wb_ref[...], dn,
                            preferred_element_type=jnp.float32)
    gamma_ref[...] = g0_ref[...] + jnp.dot(
        oh, g_all, preferred_element_type=jnp.float32)
    beta_ref[...] = b0_ref[...] + jnp.dot(
        oh, b_all, preferred_element_type=jnp.float32)


def _make_sc_embed(nb, s, d, npad):
    nw = NC * NS
    spw = s // nw
    ch = spw // 2
    nch = 2 * nb
    nin = 2
    nout = 2
    ng = d // L
    nsec = ng // SECG
    mesh = plsc.VectorSubcoreMesh(core_axis_name="c", subcore_axis_name="s")

    @functools.partial(
        pl.kernel,
        mesh=mesh,
        out_type=jax.ShapeDtypeStruct((nb * s, d), jnp.float32),
        scratch_types=[
            pltpu.VMEM((nb * spw,), jnp.int32),
            pltpu.VMEM((nin, ch, d), jnp.float32),
            pltpu.VMEM((nout, ch, d), jnp.float32),
            pltpu.VMEM((ch, d), jnp.float32),
            pltpu.VMEM((nb, d), jnp.float32),
            pltpu.VMEM((nb, d), jnp.float32),
            pltpu.SemaphoreType.DMA,
            pltpu.SemaphoreType.DMA,
            pltpu.SemaphoreType.DMA,
            pltpu.SemaphoreType.DMA,
            pltpu.SemaphoreType.DMA,
            pltpu.SemaphoreType.DMA,
            pltpu.SemaphoreType.DMA,
        ],
    )
    def sc_embed(ids_hbm, word_hbm, pos_hbm, gamma_hbm, beta_hbm, out_hbm,
                 ids_v, rin_v, rout_v, pos_v, gam_v, bet_v,
                 g0, g1, g2, o0, o1, isem, psem):
        wid = lax.axis_index("s") * NC + lax.axis_index("c")
        p0 = wid * spw
        id_cps = [pltpu.async_copy(ids_hbm.at[pl.ds(b * s + p0, spw)],
                                   ids_v.at[pl.ds(b * spw, spw)], isem)
                  for b in range(nb)]
        p_cps = [
            pltpu.async_copy(pos_hbm.at[pl.ds(p0, ch)], pos_v, psem),
            pltpu.async_copy(gamma_hbm.at[pl.ds(0, nb)], gam_v, psem),
            pltpu.async_copy(beta_hbm.at[pl.ds(0, nb)], bet_v, psem),
        ]
        for cp in id_cps:
            cp.wait()

        gsem = (g0, g1, g2)
        osem = (o0, o1)

        def start_gather(c):
            si, b = c // nb, c % nb
            return pltpu.async_copy(
                word_hbm.at[ids_v.at[pl.ds(b * spw + si * ch, ch)]],
                rin_v.at[c % nin], gsem[c % nin])

        gathers = [None] * nch
        outs = [None] * nch
        for c in range(nin - 1):
            gathers[c] = start_gather(c)
        for cp in p_cps:
            cp.wait()
        for c in range(nch):
            si, b = c // nb, c % nb
            if c == nb:
                pltpu.sync_copy(pos_hbm.at[pl.ds(p0 + ch, ch)], pos_v)
            if c + nin - 1 < nch:
                gathers[c + nin - 1] = start_gather(c + nin - 1)
            gathers[c].wait()
            if c >= nout:
                outs[c - nout].wait()
            rbuf = rin_v.at[c % nin]
            obuf = rout_v.at[c % nout]
            for sec in range(nsec):
                gg = [gam_v[b, pl.ds((sec * SECG + j) * L, L)]
                      for j in range(SECG)]
                bb = [bet_v[b, pl.ds((sec * SECG + j) * L, L)]
                      for j in range(SECG)]

                def body(r, carry, _sec=sec, _gg=gg, _bb=bb,
                         _rbuf=rbuf, _obuf=obuf):
                    for j in range(SECG):
                        sl = pl.ds((_sec * SECG + j) * L, L)
                        _obuf[r, sl] = (
                            (_rbuf[r, sl] + pos_v[r, sl])
                            * _gg[j] + _bb[j])
                    return carry

                lax.fori_loop(0, ch, body, 0)
            outs[c] = pltpu.async_copy(
                obuf, out_hbm.at[pl.ds(b * s + p0 + si * ch, ch)],
                osem[c % nout])
        for c in range(nch - nout, nch):
            outs[c].wait()

    return sc_embed


def kernel(input_ids, condition_ids, word_emb, pos_emb, cond_emb, W_hidden,
           W_gma, W_bta, clngma, clnbta):
    B, S = input_ids.shape
    vocab, D = word_emb.shape
    cond_size = cond_emb.shape[0]
    npad = 8

    gamma, beta = pl.pallas_call(
        _gamma_beta_body,
        in_specs=[
            pl.BlockSpec(memory_space=pltpu.SMEM),
            pl.BlockSpec(memory_space=pltpu.VMEM),
            pl.BlockSpec(memory_space=pltpu.VMEM),
            pl.BlockSpec(memory_space=pltpu.VMEM),
            pl.BlockSpec(memory_space=pltpu.VMEM),
            pl.BlockSpec(memory_space=pltpu.VMEM),
            pl.BlockSpec(memory_space=pltpu.VMEM),
        ],
        out_shape=(
            jax.ShapeDtypeStruct((npad, D), jnp.float32),
            jax.ShapeDtypeStruct((npad, D), jnp.float32),
        ),
    )(condition_ids.astype(jnp.int32), cond_emb, W_hidden, W_gma, W_bta,
      clngma.reshape(1, D), clnbta.reshape(1, D))

    ids_flat = input_ids.reshape(B * S).astype(jnp.int32)
    sc_embed = _make_sc_embed(B, S, D, npad)
    out = sc_embed(ids_flat, word_emb, pos_emb, gamma, beta)
    return out.reshape(B, S, D)


# speedup vs baseline: 2.3743x; 1.0022x over previous
import functools

import jax
import jax.numpy as jnp
from jax import lax
from jax.experimental import pallas as pl
from jax.experimental.pallas import tpu as pltpu
from jax.experimental.pallas import tpu_sc as plsc

NC = 2
NS = 16
L = 16
SECG = 12


def _gamma_beta_body(cids_ref, cond_ref, wh_ref, wg_ref, wb_ref, g0_ref,
                     b0_ref, gamma_ref, beta_ref):
    nb = cids_ref.shape[0]
    npad = 8
    cond_p = jnp.concatenate(
        [cond_ref[...],
         jnp.zeros((npad - cond_ref.shape[0], cond_ref.shape[1]),
                   jnp.float32)], axis=0)
    row = lax.broadcasted_iota(jnp.int32, (npad, npad), 0)
    col = lax.broadcasted_iota(jnp.int32, (npad, npad), 1)
    oh = jnp.zeros((npad, npad), jnp.float32)
    for b in range(nb):
        oh = oh + jnp.where((row == b) & (col == cids_ref[b]), 1.0, 0.0)
    dn = (((1,), (1,)), ((), ()))
    t1 = lax.dot_general(cond_p, wh_ref[...], dn,
                         preferred_element_type=jnp.float32)
    g_all = lax.dot_general(t1, wg_ref[...], dn,
                            preferred_element_type=jnp.float32)
    b_all = lax.dot_general(t1, wb_ref[...], dn,
                            preferred_element_type=jnp.float32)
    gamma_ref[...] = g0_ref[...] + jnp.dot(
        oh, g_all, preferred_element_type=jnp.float32)
    beta_ref[...] = b0_ref[...] + jnp.dot(
        oh, b_all, preferred_element_type=jnp.float32)


def _make_sc_embed(nb, s, d, npad):
    nw = NC * NS
    spw = s // nw
    ch = spw // 2
    nch = 2 * nb
    nin = 2
    nout = 2
    ng = d // L
    nsec = ng // SECG
    mesh = plsc.VectorSubcoreMesh(core_axis_name="c", subcore_axis_name="s")

    @functools.partial(
        pl.kernel,
        mesh=mesh,
        out_type=jax.ShapeDtypeStruct((nb * s, d), jnp.float32),
        scratch_types=[
            pltpu.VMEM((nb * spw,), jnp.int32),
            pltpu.VMEM((nin, ch, d), jnp.float32),
            pltpu.VMEM((nout, ch, d), jnp.float32),
            pltpu.VMEM((ch, d), jnp.float32),
            pltpu.VMEM((nb, d), jnp.float32),
            pltpu.VMEM((nb, d), jnp.float32),
            pltpu.SemaphoreType.DMA,
            pltpu.SemaphoreType.DMA,
            pltpu.SemaphoreType.DMA,
            pltpu.SemaphoreType.DMA,
            pltpu.SemaphoreType.DMA,
            pltpu.SemaphoreType.DMA,
            pltpu.SemaphoreType.DMA,
        ],
    )
    def sc_embed(ids_hbm, word_hbm, pos_hbm, gamma_hbm, beta_hbm, out_hbm,
                 ids_v, rin_v, rout_v, pos_v, gam_v, bet_v,
                 g0, g1, g2, o0, o1, isem, psem):
        wid = lax.axis_index("s") * NC + lax.axis_index("c")
        p0 = wid * spw
        id_cps = [pltpu.async_copy(ids_hbm.at[pl.ds(b * s + p0, spw)],
                                   ids_v.at[pl.ds(b * spw, spw)], isem)
                  for b in range(nb)]
        p_cps = [
            pltpu.async_copy(pos_hbm.at[pl.ds(p0, ch)], pos_v, psem),
            pltpu.async_copy(gamma_hbm.at[pl.ds(0, nb)], gam_v, psem),
            pltpu.async_copy(beta_hbm.at[pl.ds(0, nb)], bet_v, psem),
        ]

        gsem = (g0, g1, g2)
        osem = (o0, o1)
        ids_waited = [False] * nb

        def start_gather(c):
            si, b = c // nb, c % nb
            if not ids_waited[b]:
                id_cps[b].wait()
                ids_waited[b] = True
            return pltpu.async_copy(
                word_hbm.at[ids_v.at[pl.ds(b * spw + si * ch, ch)]],
                rin_v.at[c % nin], gsem[c % nin])

        gathers = [None] * nch
        outs = [None] * nch
        for c in range(nin - 1):
            gathers[c] = start_gather(c)
        for cp in p_cps:
            cp.wait()
        pos2_cp = None
        for c in range(nch):
            si, b = c // nb, c % nb
            if c == nb:
                pos2_cp.wait()
            if c + nin - 1 < nch:
                gathers[c + nin - 1] = start_gather(c + nin - 1)
            gathers[c].wait()
            if c >= nout:
                outs[c - nout].wait()
            rbuf = rin_v.at[c % nin]
            obuf = rout_v.at[c % nout]
            for sec in range(nsec):
                gg = [gam_v[b, pl.ds((sec * SECG + j) * L, L)]
                      for j in range(SECG)]
                bb = [bet_v[b, pl.ds((sec * SECG + j) * L, L)]
                      for j in range(SECG)]

                def body(r, carry, _sec=sec, _gg=gg, _bb=bb,
                         _rbuf=rbuf, _obuf=obuf):
                    for j in range(SECG):
                        sl = pl.ds((_sec * SECG + j) * L, L)
                        _obuf[r, sl] = (
                            (_rbuf[r, sl] + pos_v[r, sl])
                            * _gg[j] + _bb[j])
                    return carry

                lax.fori_loop(0, ch, body, 0)
            if c == nb - 1:
                pos2_cp = pltpu.async_copy(
                    pos_hbm.at[pl.ds(p0 + ch, ch)], pos_v, psem)
            outs[c] = pltpu.async_copy(
                obuf, out_hbm.at[pl.ds(b * s + p0 + si * ch, ch)],
                osem[c % nout])
        for c in range(nch - nout, nch):
            outs[c].wait()

    return sc_embed


def kernel(input_ids, condition_ids, word_emb, pos_emb, cond_emb, W_hidden,
           W_gma, W_bta, clngma, clnbta):
    B, S = input_ids.shape
    vocab, D = word_emb.shape
    cond_size = cond_emb.shape[0]
    npad = 8

    gamma, beta = pl.pallas_call(
        _gamma_beta_body,
        in_specs=[
            pl.BlockSpec(memory_space=pltpu.SMEM),
            pl.BlockSpec(memory_space=pltpu.VMEM),
            pl.BlockSpec(memory_space=pltpu.VMEM),
            pl.BlockSpec(memory_space=pltpu.VMEM),
            pl.BlockSpec(memory_space=pltpu.VMEM),
            pl.BlockSpec(memory_space=pltpu.VMEM),
            pl.BlockSpec(memory_space=pltpu.VMEM),
        ],
        out_shape=(
            jax.ShapeDtypeStruct((npad, D), jnp.float32),
            jax.ShapeDtypeStruct((npad, D), jnp.float32),
        ),
    )(condition_ids.astype(jnp.int32), cond_emb, W_hidden, W_gma, W_bta,
      clngma.reshape(1, D), clnbta.reshape(1, D))

    ids_flat = input_ids.reshape(B * S).astype(jnp.int32)
    sc_embed = _make_sc_embed(B, S, D, npad)
    out = sc_embed(ids_flat, word_emb, pos_emb, gamma, beta)
    return out.reshape(B, S, D)
